# baseline (speedup 1.0000x reference)
import functools

import jax
import jax.numpy as jnp
from jax import lax
from jax.experimental import pallas as pl
from jax.experimental.pallas import tpu as pltpu
from jax.experimental.pallas import tpu_sc as plsc

E = 8
H = 1024
F = 1408
T = 2048
BT = 256
NB = 2 * T // BT + E
C = NB * BT
BF = 128
NC = 2
NS = 16
NW = NC * NS
TPW = T // NW

_NEG = -1e30


def _cumsum0(a):
    n = a.shape[0]
    k = 1
    while k < n:
        shifted = jnp.concatenate(
            [jnp.zeros((k, a.shape[1]), a.dtype), a[:-k]], axis=0)
        a = a + shifted
        k *= 2
    return a


def _router_body(hs_ref, gw_ref, logits_ref, hs16_ref, pos_ref, topw_ref,
                 be_ref, bv_ref):
    hs = hs_ref[...]
    logits = jax.lax.dot_general(
        hs, gw_ref[...], (((1,), (1,)), ((), ())),
        preferred_element_type=jnp.float32)
    logits_ref[...] = logits
    hs16_ref[...] = hs.astype(jnp.bfloat16)

    lanes = lax.broadcasted_iota(jnp.int32, (T, E), 1)
    m1 = jnp.max(logits, axis=1, keepdims=True)
    i1 = jnp.min(jnp.where(logits == m1, lanes, E), axis=1, keepdims=True)
    masked = jnp.where(lanes == i1, _NEG, logits)
    m2 = jnp.max(masked, axis=1, keepdims=True)
    i2 = jnp.min(jnp.where(masked == m2, lanes, E), axis=1, keepdims=True)
    s1 = 1.0 / (1.0 + jnp.exp(m2 - m1))
    s2 = 1.0 - s1
    topw_ref[...] = jnp.concatenate([s1, s2], axis=1)

    oh1 = (lanes == i1).astype(jnp.float32)
    oh2 = (lanes == i2).astype(jnp.float32)
    oh = jnp.concatenate([oh1, oh2], axis=0)
    csum = _cumsum0(oh)
    counts = csum[2 * T - 1:2 * T, :]
    nb = jnp.floor((counts + (BT - 1)) / BT)
    lt = (lax.broadcasted_iota(jnp.int32, (E, E), 0)
          < lax.broadcasted_iota(jnp.int32, (E, E), 1)).astype(jnp.float32)
    off_blk = jax.lax.dot_general(
        nb, lt, (((1,), (0,)), ((), ())),
        preferred_element_type=jnp.float32)

    off_i = jnp.sum(oh * off_blk, axis=1, keepdims=True)
    rank_i = jnp.sum(oh * csum, axis=1, keepdims=True) - 1
    posf = off_i * BT + rank_i
    pos_ref[...] = jnp.concatenate(
        [posf[:T], posf[T:]], axis=1).astype(jnp.int32)

    b_iota = lax.broadcasted_iota(jnp.int32, (1, NB), 1).astype(jnp.float32)
    be = jnp.zeros((1, NB), jnp.float32)
    for e in range(E):
        lo = off_blk[:, e:e + 1]
        ind = jnp.logical_and(b_iota >= lo, b_iota < lo + nb[:, e:e + 1])
        be = be + e * ind.astype(jnp.float32)
    total = off_blk[:, E - 1:E] + nb[:, E - 1:E]
    bv = b_iota < total
    be_ref[...] = jnp.where(bv, be, float(E - 1)).astype(jnp.int32)
    bv_ref[...] = bv.astype(jnp.int32)


def _router(hs, gate_w):
    return pl.pallas_call(
        _router_body,
        out_shape=(
            jax.ShapeDtypeStruct((T, E), jnp.float32),
            jax.ShapeDtypeStruct((T, H), jnp.bfloat16),
            jax.ShapeDtypeStruct((T, 2), jnp.int32),
            jax.ShapeDtypeStruct((T, 2), jnp.float32),
            jax.ShapeDtypeStruct((1, NB), jnp.int32),
            jax.ShapeDtypeStruct((1, NB), jnp.int32),
        ),
    )(hs, gate_w)


@functools.cache
def _sc_dispatch():
    mesh = plsc.VectorSubcoreMesh(core_axis_name="c", subcore_axis_name="s")

    @functools.partial(
        pl.kernel,
        mesh=mesh,
        out_type=jax.ShapeDtypeStruct((C, H // 2), jnp.int32),
        scratch_types=[
            pltpu.VMEM((4, 32), jnp.int32),
            pltpu.VMEM((TPW, H // 2), jnp.int32),
            pltpu.SemaphoreType.DMA,
        ],
    )
    def _dispatch(hs16_hbm, psc_hbm, out_hbm, idx_vm, rows_vm, sem):
        wid = lax.axis_index("s") * NC + lax.axis_index("c")
        pltpu.sync_copy(psc_hbm.at[wid], idx_vm)
        pltpu.sync_copy(hs16_hbm.at[pl.ds(wid * TPW, TPW)], rows_vm)
        for j in range(2):
            for h in range(2):
                pltpu.async_copy(
                    rows_vm.at[pl.ds(h * 32, 32)],
                    out_hbm.at[idx_vm.at[j * 2 + h]],
                    sem,
                ).wait()

    return _dispatch


def _gmm_body(be_ref, bv_ref, x_ref, wg_ref, wu_ref, wd_ref, out_ref):
    f = pl.program_id(0)
    b = pl.program_id(1)

    @pl.when(bv_ref[b] == 1)
    def _():
        xb = x_ref[pl.ds(b * BT, BT), :]
        wg = wg_ref[0].astype(jnp.bfloat16)
        wu = wu_ref[0].astype(jnp.bfloat16)
        wd = wd_ref[0].astype(jnp.bfloat16)
        g = jax.lax.dot_general(
            xb, wg, (((1,), (1,)), ((), ())),
            preferred_element_type=jnp.float32)
        u = jax.lax.dot_general(
            xb, wu, (((1,), (1,)), ((), ())),
            preferred_element_type=jnp.float32)
        act = g * (1.0 / (1.0 + jnp.exp(-g))) * u
        contrib = jax.lax.dot_general(
            act.astype(jnp.bfloat16), wd, (((1,), (1,)), ((), ())),
            preferred_element_type=jnp.float32)

        @pl.when(f == 0)
        def _():
            out_ref[pl.ds(b * BT, BT), :] = contrib

        @pl.when(f > 0)
        def _():
            out_ref[pl.ds(b * BT, BT), :] += contrib


def _gmm(be, bv, x16, gpw, upw, dpw):
    grid_spec = pltpu.PrefetchScalarGridSpec(
        num_scalar_prefetch=2,
        grid=(F // BF, NB),
        in_specs=[
            pl.BlockSpec((C, H), lambda f, b, be, bv: (0, 0)),
            pl.BlockSpec((1, BF, H), lambda f, b, be, bv: (be[b], f, 0)),
            pl.BlockSpec((1, BF, H), lambda f, b, be, bv: (be[b], f, 0)),
            pl.BlockSpec((1, H, BF), lambda f, b, be, bv: (be[b], 0, f)),
        ],
        out_specs=pl.BlockSpec((C, H), lambda f, b, be, bv: (0, 0)),
    )
    return pl.pallas_call(
        _gmm_body,
        grid_spec=grid_spec,
        out_shape=jax.ShapeDtypeStruct((C, H), jnp.float32),
    )(be, bv, x16, gpw, upw, dpw)


@functools.cache
def _sc_combine():
    mesh = plsc.VectorSubcoreMesh(core_axis_name="c", subcore_axis_name="s")

    @functools.partial(
        pl.kernel,
        mesh=mesh,
        out_type=jax.ShapeDtypeStruct((T, H), jnp.float32),
        scratch_types=[
            pltpu.VMEM((4, 32), jnp.int32),
            pltpu.VMEM((4, 32, 16), jnp.float32),
            pltpu.VMEM((32, H), jnp.float32),
            pltpu.VMEM((32, H), jnp.float32),
            pltpu.SemaphoreType.DMA,
        ],
    )
    def _combine(rows_hbm, psc_hbm, wsc_hbm, out_hbm, idx_vm, w_vm, b0, b1,
                 sem):
        wid = lax.axis_index("s") * NC + lax.axis_index("c")
        pltpu.sync_copy(psc_hbm.at[wid], idx_vm)
        pltpu.sync_copy(wsc_hbm.at[wid], w_vm)
        for h in range(2):
            pltpu.async_copy(rows_hbm.at[idx_vm.at[h]], b0, sem).wait()
            pltpu.async_copy(rows_hbm.at[idx_vm.at[2 + h]], b1, sem).wait()

            def row_body(r, carry):
                sv0 = w_vm[h, r]
                sv1 = w_vm[2 + h, r]
                for cc in range(H // 16):
                    sl = pl.ds(cc * 16, 16)
                    b0[r, sl] = sv0 * b0[r, sl] + sv1 * b1[r, sl]
                return carry

            lax.fori_loop(0, 32, row_body, 0)
            pltpu.sync_copy(b0, out_hbm.at[pl.ds(wid * TPW + h * 32, 32)])

    return _combine


def kernel(x, gate_w, gate_proj_w, up_proj_w, down_proj_w):
    b, s, h = x.shape
    hs = x.reshape(-1, h)
    logits, hs16, pos, topw, be, bv = _router(hs, gate_w)

    psc = pos.reshape(NW, 2, 32, 2).transpose(0, 3, 1, 2).reshape(NW, 4, 32)
    wsc = jnp.broadcast_to(
        topw.reshape(NW, 2, 32, 2).transpose(0, 3, 1, 2).reshape(NW, 4, 32)[
            ..., None], (NW, 4, 32, 16)).astype(jnp.float32)

    hs_i32 = lax.bitcast_convert_type(
        hs16.reshape(T, H // 2, 2), jnp.int32)
    sorted_i32 = _sc_dispatch()(hs_i32, psc)
    sorted_x = lax.bitcast_convert_type(
        sorted_i32, jnp.bfloat16).reshape(C, H)
    rows = _gmm(be.reshape(NB), bv.reshape(NB), sorted_x,
                gate_proj_w, up_proj_w, down_proj_w)
    final = _sc_combine()(rows, psc, wsc)
    return final.reshape(b, s, h), logits


# speedup vs baseline: 3.1430x; 3.1430x over previous
import functools

import jax
import jax.numpy as jnp
from jax import lax
from jax.experimental import pallas as pl
from jax.experimental.pallas import tpu as pltpu
from jax.experimental.pallas import tpu_sc as plsc

E = 8
H = 1024
F = 1408
T = 2048
BT = 256
NB = 2 * T // BT + E
C = NB * BT
NC = 2
NS = 16
NW = NC * NS
TPW = T // NW

_NEG = -1e30


def _cumsum0(a):
    n = a.shape[0]
    k = 1
    while k < n:
        shifted = jnp.concatenate(
            [jnp.zeros((k, a.shape[1]), a.dtype), a[:-k]], axis=0)
        a = a + shifted
        k *= 2
    return a


def _router_body(hs_ref, gw_ref, logits_ref, pos_ref, wsc_ref, be_ref,
                 bv_ref):
    hs = hs_ref[...]
    logits = jax.lax.dot_general(
        hs, gw_ref[...], (((1,), (1,)), ((), ())),
        preferred_element_type=jnp.float32)
    logits_ref[...] = logits

    lanes = lax.broadcasted_iota(jnp.int32, (T, E), 1)
    m1 = jnp.max(logits, axis=1, keepdims=True)
    i1 = jnp.min(jnp.where(logits == m1, lanes, E), axis=1, keepdims=True)
    masked = jnp.where(lanes == i1, _NEG, logits)
    m2 = jnp.max(masked, axis=1, keepdims=True)
    i2 = jnp.min(jnp.where(masked == m2, lanes, E), axis=1, keepdims=True)
    s1 = 1.0 / (1.0 + jnp.exp(m2 - m1))
    s2 = 1.0 - s1
    wsc_ref[0] = jnp.broadcast_to(s1, (T, 16))
    wsc_ref[1] = jnp.broadcast_to(s2, (T, 16))

    oh1 = (lanes == i1).astype(jnp.float32)
    oh2 = (lanes == i2).astype(jnp.float32)
    oh = jnp.concatenate([oh1, oh2], axis=0)
    csum = _cumsum0(oh)
    counts = csum[2 * T - 1:2 * T, :]
    nb = jnp.floor((counts + (BT - 1)) / BT)
    lt = (lax.broadcasted_iota(jnp.int32, (E, E), 0)
          < lax.broadcasted_iota(jnp.int32, (E, E), 1)).astype(jnp.float32)
    off_blk = jax.lax.dot_general(
        nb, lt, (((1,), (0,)), ((), ())),
        preferred_element_type=jnp.float32)

    off_i = jnp.sum(oh * off_blk, axis=1, keepdims=True)
    rank_i = jnp.sum(oh * csum, axis=1, keepdims=True) - 1
    posf = off_i * BT + rank_i
    pos_ref[...] = jnp.concatenate(
        [posf[:T], posf[T:]], axis=1).astype(jnp.int32)

    b_iota = lax.broadcasted_iota(jnp.int32, (1, NB), 1).astype(jnp.float32)
    be = jnp.zeros((1, NB), jnp.float32)
    for e in range(E):
        lo = off_blk[:, e:e + 1]
        ind = jnp.logical_and(b_iota >= lo, b_iota < lo + nb[:, e:e + 1])
        be = be + e * ind.astype(jnp.float32)
    total = off_blk[:, E - 1:E] + nb[:, E - 1:E]
    bv = b_iota < total
    be_ref[...] = jnp.where(bv, be, float(E - 1)).astype(jnp.int32)
    bv_ref[...] = bv.astype(jnp.int32)


def _router(hs, gate_w):
    return pl.pallas_call(
        _router_body,
        out_shape=(
            jax.ShapeDtypeStruct((T, E), jnp.float32),
            jax.ShapeDtypeStruct((T, 2), jnp.int32),
            jax.ShapeDtypeStruct((2, T, 16), jnp.float32),
            jax.ShapeDtypeStruct((1, NB), jnp.int32),
            jax.ShapeDtypeStruct((1, NB), jnp.int32),
        ),
    )(hs, gate_w)


@functools.cache
def _sc_dispatch():
    mesh = plsc.VectorSubcoreMesh(core_axis_name="c", subcore_axis_name="s")

    @functools.partial(
        pl.kernel,
        mesh=mesh,
        out_type=jax.ShapeDtypeStruct((C, H), jnp.float32),
        scratch_types=[
            pltpu.VMEM((2, TPW), jnp.int32),
            pltpu.VMEM((TPW, H), jnp.float32),
            pltpu.SemaphoreType.DMA,
        ],
    )
    def _dispatch(hs_hbm, psc_hbm, out_hbm, idx_vm, rows_vm, sem):
        wid = lax.axis_index("s") * NC + lax.axis_index("c")
        for j in range(2):
            pltpu.sync_copy(psc_hbm.at[j, pl.ds(wid * TPW, TPW)],
                            idx_vm.at[j])
        pltpu.sync_copy(hs_hbm.at[pl.ds(wid * TPW, TPW)], rows_vm)
        for j in range(2):
            pltpu.async_copy(
                rows_vm, out_hbm.at[idx_vm.at[j]], sem).wait()

    return _dispatch


def _gmm_body(be_ref, bv_ref, x_ref, wg_ref, wu_ref, wd_ref, out_ref):
    b = pl.program_id(0)

    @pl.when(bv_ref[b] == 1)
    def _():
        xb = x_ref[...].astype(jnp.bfloat16)
        wg = wg_ref[0].astype(jnp.bfloat16)
        wu = wu_ref[0].astype(jnp.bfloat16)
        wd = wd_ref[0].astype(jnp.bfloat16)
        g = jax.lax.dot_general(
            xb, wg, (((1,), (1,)), ((), ())),
            preferred_element_type=jnp.float32)
        u = jax.lax.dot_general(
            xb, wu, (((1,), (1,)), ((), ())),
            preferred_element_type=jnp.float32)
        act = g * (1.0 / (1.0 + jnp.exp(-g))) * u
        out_ref[...] = jax.lax.dot_general(
            act.astype(jnp.bfloat16), wd, (((1,), (1,)), ((), ())),
            preferred_element_type=jnp.float32)


def _gmm(be, bv, sorted_x, gpw, upw, dpw):
    grid_spec = pltpu.PrefetchScalarGridSpec(
        num_scalar_prefetch=2,
        grid=(NB,),
        in_specs=[
            pl.BlockSpec((BT, H), lambda b, be, bv: (b, 0)),
            pl.BlockSpec((1, F, H), lambda b, be, bv: (be[b], 0, 0)),
            pl.BlockSpec((1, F, H), lambda b, be, bv: (be[b], 0, 0)),
            pl.BlockSpec((1, H, F), lambda b, be, bv: (be[b], 0, 0)),
        ],
        out_specs=pl.BlockSpec((BT, H), lambda b, be, bv: (b, 0)),
    )
    return pl.pallas_call(
        _gmm_body,
        grid_spec=grid_spec,
        out_shape=jax.ShapeDtypeStruct((C, H), jnp.float32),
    )(be, bv, sorted_x, gpw, upw, dpw)


@functools.cache
def _sc_combine():
    mesh = plsc.VectorSubcoreMesh(core_axis_name="c", subcore_axis_name="s")

    @functools.partial(
        pl.kernel,
        mesh=mesh,
        out_type=jax.ShapeDtypeStruct((T, H), jnp.float32),
        scratch_types=[
            pltpu.VMEM((2, TPW), jnp.int32),
            pltpu.VMEM((2, TPW, 16), jnp.float32),
            pltpu.VMEM((TPW // 2, H), jnp.float32),
            pltpu.VMEM((TPW // 2, H), jnp.float32),
            pltpu.SemaphoreType.DMA,
        ],
    )
    def _combine(rows_hbm, psc_hbm, wsc_hbm, out_hbm, idx_vm, w_vm, b0, b1,
                 sem):
        wid = lax.axis_index("s") * NC + lax.axis_index("c")
        for j in range(2):
            pltpu.sync_copy(psc_hbm.at[j, pl.ds(wid * TPW, TPW)],
                            idx_vm.at[j])
            pltpu.sync_copy(wsc_hbm.at[j, pl.ds(wid * TPW, TPW)],
                            w_vm.at[j])
        for h in range(2):
            pltpu.async_copy(
                rows_hbm.at[idx_vm.at[0, pl.ds(h * 32, 32)]], b0, sem).wait()
            pltpu.async_copy(
                rows_hbm.at[idx_vm.at[1, pl.ds(h * 32, 32)]], b1, sem).wait()

            def row_body(r, carry):
                sv0 = w_vm[0, h * 32 + r]
                sv1 = w_vm[1, h * 32 + r]
                for cc in range(H // 16):
                    sl = pl.ds(cc * 16, 16)
                    b0[r, sl] = sv0 * b0[r, sl] + sv1 * b1[r, sl]
                return carry

            lax.fori_loop(0, TPW // 2, row_body, 0)
            pltpu.sync_copy(b0, out_hbm.at[pl.ds(wid * TPW + h * 32, 32)])

    return _combine


def kernel(x, gate_w, gate_proj_w, up_proj_w, down_proj_w):
    b, s, h = x.shape
    hs = x.reshape(-1, h)
    logits, pos, wsc, be, bv = _router(hs, gate_w)

    psc = pos.T
    sorted_x = _sc_dispatch()(hs, psc)
    rows = _gmm(be.reshape(NB), bv.reshape(NB), sorted_x,
                gate_proj_w, up_proj_w, down_proj_w)
    final = _sc_combine()(rows, psc, wsc)
    return final.reshape(b, s, h), logits


# speedup vs baseline: 3.1969x; 1.0171x over previous
import functools

import jax
import jax.numpy as jnp
from jax import lax
from jax.experimental import pallas as pl
from jax.experimental.pallas import tpu as pltpu
from jax.experimental.pallas import tpu_sc as plsc

E = 8
H = 1024
F = 1408
T = 2048
BT = 256
NB = 2 * T // BT + E
C = NB * BT
NC = 2
NS = 16
NW = NC * NS
TPW = T // NW

_NEG = -1e30


def _cumsum0(a):
    n = a.shape[0]
    k = 1
    while k < n:
        shifted = jnp.concatenate(
            [jnp.zeros((k, a.shape[1]), a.dtype), a[:-k]], axis=0)
        a = a + shifted
        k *= 2
    return a


def _router_body(hs_ref, gw_ref, logits_ref, pos_ref, wsc_ref, be_ref,
                 bv_ref):
    hs = hs_ref[...]
    logits = jax.lax.dot_general(
        hs, gw_ref[...], (((1,), (1,)), ((), ())),
        preferred_element_type=jnp.float32)
    logits_ref[...] = logits

    lanes = lax.broadcasted_iota(jnp.int32, (T, E), 1)
    m1 = jnp.max(logits, axis=1, keepdims=True)
    i1 = jnp.min(jnp.where(logits == m1, lanes, E), axis=1, keepdims=True)
    masked = jnp.where(lanes == i1, _NEG, logits)
    m2 = jnp.max(masked, axis=1, keepdims=True)
    i2 = jnp.min(jnp.where(masked == m2, lanes, E), axis=1, keepdims=True)
    s1 = 1.0 / (1.0 + jnp.exp(m2 - m1))
    s2 = 1.0 - s1
    wsc_ref[0] = jnp.broadcast_to(s1, (T, 16))
    wsc_ref[1] = jnp.broadcast_to(s2, (T, 16))

    oh1 = (lanes == i1).astype(jnp.float32)
    oh2 = (lanes == i2).astype(jnp.float32)
    oh = jnp.concatenate([oh1, oh2], axis=0)
    csum = _cumsum0(oh)
    counts = csum[2 * T - 1:2 * T, :]
    nb = jnp.floor((counts + (BT - 1)) / BT)
    lt = (lax.broadcasted_iota(jnp.int32, (E, E), 0)
          < lax.broadcasted_iota(jnp.int32, (E, E), 1)).astype(jnp.float32)
    off_blk = jax.lax.dot_general(
        nb, lt, (((1,), (0,)), ((), ())),
        preferred_element_type=jnp.float32)

    off_i = jnp.sum(oh * off_blk, axis=1, keepdims=True)
    rank_i = jnp.sum(oh * csum, axis=1, keepdims=True) - 1
    posf = off_i * BT + rank_i
    pos_ref[...] = jnp.concatenate(
        [posf[:T], posf[T:]], axis=1).astype(jnp.int32)

    b_iota = lax.broadcasted_iota(jnp.int32, (1, NB), 1).astype(jnp.float32)
    be = jnp.zeros((1, NB), jnp.float32)
    for e in range(E):
        lo = off_blk[:, e:e + 1]
        ind = jnp.logical_and(b_iota >= lo, b_iota < lo + nb[:, e:e + 1])
        be = be + e * ind.astype(jnp.float32)
    total = off_blk[:, E - 1:E] + nb[:, E - 1:E]
    bv = b_iota < total
    be_ref[...] = jnp.where(bv, be, float(E - 1)).astype(jnp.int32)
    bv_ref[...] = bv.astype(jnp.int32)


def _router(hs, gate_w):
    return pl.pallas_call(
        _router_body,
        out_shape=(
            jax.ShapeDtypeStruct((T, E), jnp.float32),
            jax.ShapeDtypeStruct((T, 2), jnp.int32),
            jax.ShapeDtypeStruct((2, T, 16), jnp.float32),
            jax.ShapeDtypeStruct((1, NB), jnp.int32),
            jax.ShapeDtypeStruct((1, NB), jnp.int32),
        ),
    )(hs, gate_w)


@functools.cache
def _sc_dispatch():
    mesh = plsc.VectorSubcoreMesh(core_axis_name="c", subcore_axis_name="s")

    @functools.partial(
        pl.kernel,
        mesh=mesh,
        out_type=jax.ShapeDtypeStruct((C, H), jnp.float32),
    scratch_types=[
            pltpu.VMEM((2, TPW), jnp.int32),
            pltpu.VMEM((TPW, H), jnp.float32),
            pltpu.SemaphoreType.DMA,
            pltpu.SemaphoreType.DMA,
        ],
    )
    def _dispatch(hs_hbm, psc_hbm, out_hbm, idx_vm, rows_vm, sem0, sem1):
        wid = lax.axis_index("s") * NC + lax.axis_index("c")
        for j in range(2):
            pltpu.sync_copy(psc_hbm.at[j, pl.ds(wid * TPW, TPW)],
                            idx_vm.at[j])
        pltpu.sync_copy(hs_hbm.at[pl.ds(wid * TPW, TPW)], rows_vm)
        cp0 = pltpu.async_copy(rows_vm, out_hbm.at[idx_vm.at[0]], sem0)
        cp1 = pltpu.async_copy(rows_vm, out_hbm.at[idx_vm.at[1]], sem1)
        cp0.wait()
        cp1.wait()

    return _dispatch


def _gmm_body(be_ref, bv_ref, x_ref, wg_ref, wu_ref, wd_ref, out_ref):
    b = pl.program_id(0)

    @pl.when(bv_ref[b] == 1)
    def _():
        xb = x_ref[...].astype(jnp.bfloat16)
        wg = wg_ref[0].astype(jnp.bfloat16)
        wu = wu_ref[0].astype(jnp.bfloat16)
        wd = wd_ref[0].astype(jnp.bfloat16)
        g = jax.lax.dot_general(
            xb, wg, (((1,), (1,)), ((), ())),
            preferred_element_type=jnp.float32)
        u = jax.lax.dot_general(
            xb, wu, (((1,), (1,)), ((), ())),
            preferred_element_type=jnp.float32)
        act = g * (1.0 / (1.0 + jnp.exp(-g))) * u
        out_ref[...] = jax.lax.dot_general(
            act.astype(jnp.bfloat16), wd, (((1,), (1,)), ((), ())),
            preferred_element_type=jnp.float32)


def _gmm(be, bv, sorted_x, gpw, upw, dpw):
    grid_spec = pltpu.PrefetchScalarGridSpec(
        num_scalar_prefetch=2,
        grid=(NB,),
        in_specs=[
            pl.BlockSpec((BT, H),
                         lambda b, be, bv: (jnp.where(bv[b] == 1, b, 0), 0)),
            pl.BlockSpec((1, F, H), lambda b, be, bv: (be[b], 0, 0)),
            pl.BlockSpec((1, F, H), lambda b, be, bv: (be[b], 0, 0)),
            pl.BlockSpec((1, H, F), lambda b, be, bv: (be[b], 0, 0)),
        ],
        out_specs=pl.BlockSpec((BT, H), lambda b, be, bv: (b, 0)),
    )
    return pl.pallas_call(
        _gmm_body,
        grid_spec=grid_spec,
        out_shape=jax.ShapeDtypeStruct((C, H), jnp.float32),
    )(be, bv, sorted_x, gpw, upw, dpw)


@functools.cache
def _sc_combine():
    mesh = plsc.VectorSubcoreMesh(core_axis_name="c", subcore_axis_name="s")

    @functools.partial(
        pl.kernel,
        mesh=mesh,
        out_type=jax.ShapeDtypeStruct((T, H), jnp.float32),
        scratch_types=[
            pltpu.VMEM((2, TPW), jnp.int32),
            pltpu.VMEM((2, TPW, 16), jnp.float32),
            pltpu.VMEM((2, TPW // 4, H), jnp.float32),
            pltpu.VMEM((2, TPW // 4, H), jnp.float32),
            pltpu.SemaphoreType.DMA,
            pltpu.SemaphoreType.DMA,
            pltpu.SemaphoreType.DMA,
            pltpu.SemaphoreType.DMA,
        ],
    )
    def _combine(rows_hbm, psc_hbm, wsc_hbm, out_hbm, idx_vm, w_vm, b0, b1,
                 s0a, s0b, s1a, s1b):
        wid = lax.axis_index("s") * NC + lax.axis_index("c")
        for j in range(2):
            pltpu.sync_copy(psc_hbm.at[j, pl.ds(wid * TPW, TPW)],
                            idx_vm.at[j])
            pltpu.sync_copy(wsc_hbm.at[j, pl.ds(wid * TPW, TPW)],
                            w_vm.at[j])
        Q = TPW // 4
        sems = ((s0a, s1a), (s0b, s1b))

        def fire(q):
            par = q % 2
            sl = pl.ds(q * Q, Q)
            return (
                pltpu.async_copy(rows_hbm.at[idx_vm.at[0, sl]], b0.at[par],
                                 sems[par][0]),
                pltpu.async_copy(rows_hbm.at[idx_vm.at[1, sl]], b1.at[par],
                                 sems[par][1]),
            )

        cps = fire(0)
        for q in range(4):
            nxt = fire(q + 1) if q + 1 < 4 else None
            cps[0].wait()
            cps[1].wait()
            par = q % 2

            def row_body(r, carry):
                sv0 = w_vm[0, q * Q + r]
                sv1 = w_vm[1, q * Q + r]
                for cc in range(H // 16):
                    sl = pl.ds(cc * 16, 16)
                    b0[par, r, sl] = (sv0 * b0[par, r, sl]
                                      + sv1 * b1[par, r, sl])
                return carry

            lax.fori_loop(0, Q, row_body, 0)
            pltpu.sync_copy(b0.at[par], out_hbm.at[pl.ds(wid * TPW + q * Q,
                                                         Q)])
            cps = nxt

    return _combine


def kernel(x, gate_w, gate_proj_w, up_proj_w, down_proj_w):
    b, s, h = x.shape
    hs = x.reshape(-1, h)
    logits, pos, wsc, be, bv = _router(hs, gate_w)

    psc = pos.T
    sorted_x = _sc_dispatch()(hs, psc)
    rows = _gmm(be.reshape(NB), bv.reshape(NB), sorted_x,
                gate_proj_w, up_proj_w, down_proj_w)
    final = _sc_combine()(rows, psc, wsc)
    return final.reshape(b, s, h), logits
